# baseline (speedup 1.0000x reference)
import functools

import jax
import jax.numpy as jnp
from jax import lax
from jax.experimental import pallas as pl
from jax.experimental.pallas import tpu as pltpu
from jax.experimental.pallas import tpu_sc as plsc

_B, _L, _D = 16, 4096, 512
_D0, _D1, _D2 = 256, 153, 103
_P1, _P2 = 256, 128


def _sc_gather(E0, E1p, E2p, gh0, gh1, gh2):
    info = plsc.get_sparse_core_info()
    nc, ns = info.num_cores, info.num_subcores
    nw = nc * ns
    bpw = _L // nw
    mesh = plsc.VectorSubcoreMesh(core_axis_name="c", subcore_axis_name="s")

    @functools.partial(
        pl.kernel,
        mesh=mesh,
        out_type=(
            jax.ShapeDtypeStruct((_L, _D0), jnp.float32),
            jax.ShapeDtypeStruct((_L, _P1), jnp.float32),
            jax.ShapeDtypeStruct((_L, _P2), jnp.float32),
        ),
        scratch_types=[
            pltpu.VMEM((bpw,), jnp.int32),
            pltpu.VMEM((bpw,), jnp.int32),
            pltpu.VMEM((bpw,), jnp.int32),
            pltpu.VMEM((bpw, _D0), jnp.float32),
            pltpu.VMEM((bpw, _P1), jnp.float32),
            pltpu.VMEM((bpw, _P2), jnp.float32),
            pltpu.SemaphoreType.DMA,
        ],
    )
    def k(e0_h, e1_h, e2_h, g0_h, g1_h, g2_h, o0_h, o1_h, o2_h,
          i0, i1, i2, r0, r1, r2, sem):
        wid = lax.axis_index("s") * nc + lax.axis_index("c")
        base = wid * bpw
        pltpu.sync_copy(g0_h.at[pl.ds(base, bpw)], i0)
        pltpu.sync_copy(g1_h.at[pl.ds(base, bpw)], i1)
        pltpu.sync_copy(g2_h.at[pl.ds(base, bpw)], i2)
        c0 = pltpu.async_copy(e0_h.at[i0], r0, sem)
        c1 = pltpu.async_copy(e1_h.at[i1], r1, sem)
        c2 = pltpu.async_copy(e2_h.at[i2], r2, sem)
        c0.wait()
        c1.wait()
        c2.wait()
        pltpu.sync_copy(r0, o0_h.at[pl.ds(base, bpw)])
        pltpu.sync_copy(r1, o1_h.at[pl.ds(base, bpw)])
        pltpu.sync_copy(r2, o2_h.at[pl.ds(base, bpw)])

    return k(E0, E1p, E2p, gh0, gh1, gh2)


_TL = 1024


def _add_body(x_ref, e0_ref, e1_ref, e2_ref, o_ref):
    emb = jnp.concatenate(
        [e0_ref[...], e1_ref[:, : _D1], e2_ref[:, : _D2]], axis=-1
    )
    o_ref[...] = x_ref[...] + emb[None]


def _tc_add(x, e0, e1, e2):
    return pl.pallas_call(
        _add_body,
        grid=(_L // _TL, _B),
        in_specs=[
            pl.BlockSpec((1, _TL, _D), lambda l, b: (b, l, 0)),
            pl.BlockSpec((_TL, _D0), lambda l, b: (l, 0)),
            pl.BlockSpec((_TL, _P1), lambda l, b: (l, 0)),
            pl.BlockSpec((_TL, _P2), lambda l, b: (l, 0)),
        ],
        out_specs=pl.BlockSpec((1, _TL, _D), lambda l, b: (b, l, 0)),
        out_shape=jax.ShapeDtypeStruct((_B, _L, _D), jnp.float32),
        compiler_params=pltpu.CompilerParams(
            dimension_semantics=("arbitrary", "arbitrary")
        ),
    )(x, e0, e1, e2)


def kernel(x, E0, E1, E2, gh0, gh1, gh2):
    E1p = jnp.pad(E1, ((0, 0), (0, _P1 - _D1)))
    E2p = jnp.pad(E2, ((0, 0), (0, _P2 - _D2)))
    e0, e1, e2 = _sc_gather(E0, E1p, E2p, gh0, gh1, gh2)
    return _tc_add(x, e0, e1, e2)
